# baseline (speedup 1.0000x reference)
import functools

import jax
import jax.numpy as jnp
from jax import lax
from jax.experimental import pallas as pl
from jax.experimental.pallas import tpu as pltpu
from jax.experimental.pallas import tpu_sc as plsc

D = 128
NEI = 32
K = 16
B = 1024
R = 2000
NE = 50000
S = 16
RP = 2048
EW = 128
NC = 2
NS = 16
NW = NC * NS
LANES = 16

MI = (B * K) // NW
MCI = 128
MCH = MI // MCI


def _mesh():
    return plsc.VectorSubcoreMesh(core_axis_name="c", subcore_axis_name="s")


def _wid():
    return lax.axis_index("s") * NC + lax.axis_index("c")


def _dg16(vec, idx):
    return lax.gather(
        vec, idx[:, None],
        lax.GatherDimensionNumbers(
            offset_dims=(), collapsed_slice_dims=(0,), start_index_map=(0,)),
        (1,), mode=lax.GatherScatterMode.PROMISE_IN_BOUNDS)


def _make_edge_gather(C, first, with_emb):
    WB = B // NW
    NI = WB * C
    CI = min(128, NI)
    NCH = NI // CI
    BC = CI // C
    N = C * NEI

    out_type = [
        jax.ShapeDtypeStruct((B, N), jnp.float32),
        jax.ShapeDtypeStruct((B, N), jnp.int32),
    ]
    scratch = [
        pltpu.VMEM((R,), jnp.float32),
        pltpu.VMEM((NCH, CI), jnp.int32),
        pltpu.VMEM((NCH, CI), jnp.int32),
        pltpu.VMEM((NI, EW), jnp.int32),
        pltpu.VMEM((BC, N), jnp.float32),
        pltpu.VMEM((BC, N), jnp.int32),
        pltpu.SemaphoreType.DMA,
    ]
    if first:
        out_type.append(jax.ShapeDtypeStruct((R,), jnp.float32))
        scratch = [pltpu.VMEM((S,), jnp.int32),
                   pltpu.VMEM((S, R), jnp.float32)] + scratch
    if with_emb:
        out_type.append(jax.ShapeDtypeStruct((B * K, D), jnp.float32))
        scratch += [pltpu.VMEM((MCH, MCI), jnp.int32),
                    pltpu.VMEM((MCI, D), jnp.float32),
                    pltpu.VMEM((MCI, D), jnp.float32),
                    pltpu.SemaphoreType.DMA,
                    pltpu.SemaphoreType.DMA]

    @functools.partial(
        pl.kernel,
        out_type=tuple(out_type),
        mesh=_mesh(),
        compiler_params=pltpu.CompilerParams(needs_layout_passes=False),
        scratch_types=scratch,
    )
    def k(*refs):
        it = iter(refs)
        edge_hbm, cur_hbm = next(it), next(it)
        if first:
            cos_hbm, srel_hbm = next(it), next(it)
        else:
            rs_hbm = next(it)
        if with_emb:
            emtab_hbm, erel_hbm = next(it), next(it)
        osc_hbm, opk_hbm = next(it), next(it)
        if first:
            rs_hbm = next(it)
        if with_emb:
            emb_hbm = next(it)
        if first:
            srel_v, cos_v = next(it), next(it)
        tab_v, idx_v, hid_v, rows_v, osc_v, opk_v, sem = (
            next(it), next(it), next(it), next(it), next(it), next(it),
            next(it))
        if with_emb:
            midx_v, mrows0_v, mrows1_v, msem0, msem1 = (
                next(it), next(it), next(it), next(it), next(it))
        wid = _wid()

        for ch in range(NCH):
            pltpu.sync_copy(cur_hbm.at[pl.ds(wid * NI + ch * CI, CI)],
                            idx_v.at[ch])
            for jv in range(CI // LANES):
                o = pl.ds(jv * LANES, LANES)
                hid_v[ch, o] = lax.shift_right_logical(idx_v[ch, o], 1)
        ecopies = [
            pltpu.async_copy(edge_hbm.at[hid_v.at[ch]],
                             rows_v.at[pl.ds(ch * CI, CI)], sem)
            for ch in range(NCH)
        ]
        if with_emb:
            for ch in range(MCH):
                pltpu.sync_copy(erel_hbm.at[pl.ds(wid * MI + ch * MCI, MCI)],
                                midx_v.at[ch])
            mbufs = [mrows0_v, mrows1_v]
            msems = [msem0, msem1]
            mcopies = [
                pltpu.async_copy(emtab_hbm.at[midx_v.at[ch]],
                                 mbufs[ch % 2], msems[ch % 2])
                for ch in range(2)
            ]
        if first:
            pltpu.sync_copy(srel_hbm, srel_v)
            srel_vec = srel_v[...]
            slane = lax.iota(jnp.int32, LANES)
            for s in range(S):
                row = jnp.max(jnp.where(slane == s, srel_vec, 0))
                pltpu.sync_copy(cos_hbm.at[row], cos_v.at[s])

            def tbody(j, carry):
                sl = pl.ds(j * LANES, LANES)
                m = cos_v[0, sl]
                for s in range(1, S):
                    m = jnp.maximum(m, cos_v[s, sl])
                tab_v[sl] = m
                return carry

            lax.fori_loop(0, R // LANES, tbody, 0)

            @pl.when(wid == 0)
            def _():
                pltpu.sync_copy(tab_v, rs_hbm)
        else:
            pltpu.sync_copy(rs_hbm, tab_v)

        lane = lax.iota(jnp.int32, LANES)
        pat_e = (lane & 7) * 2
        pat_o = pat_e + 1
        lo = lane < 8
        for ch in range(NCH):
            ecopies[ch].wait()

            def body(brow, carry):
                for c in range(C):
                    rr = brow * C + c
                    r = ch * CI + rr
                    ev = idx_v[ch, pl.ds((rr // LANES) * LANES, LANES)]
                    par = jnp.max(jnp.where(lane == rr % LANES, ev & 1, 0))
                    base = par * (2 * NEI)
                    for v2 in range(2):
                        a = rows_v[r, pl.ds(base + v2 * 2 * LANES, LANES)]
                        b = rows_v[r, pl.ds(base + v2 * 2 * LANES + LANES,
                                            LANES)]
                        entv = jnp.where(lo, _dg16(a, pat_e), _dg16(b, pat_e))
                        relv = jnp.where(lo, _dg16(a, pat_o), _dg16(b, pat_o))
                        scv = plsc.load_gather(tab_v, [relv])
                        o = pl.ds(c * NEI + v2 * LANES, LANES)
                        osc_v[brow, o] = scv
                        opk_v[brow, o] = entv * 2048 + relv
                return carry

            lax.fori_loop(0, BC, body, 0)
            ob = wid * WB + ch * BC
            pltpu.sync_copy(osc_v, osc_hbm.at[pl.ds(ob, BC)])
            pltpu.sync_copy(opk_v, opk_hbm.at[pl.ds(ob, BC)])

        if with_emb:
            for ch in range(MCH):
                mcopies[ch].wait()
                pltpu.sync_copy(
                    mbufs[ch % 2],
                    emb_hbm.at[pl.ds(wid * MI + ch * MCI, MCI)])
                if ch + 2 < MCH:
                    mcopies.append(pltpu.async_copy(
                        emtab_hbm.at[midx_v.at[ch + 2]],
                        mbufs[ch % 2], msems[ch % 2]))

    return k


_edge_gather_1 = _make_edge_gather(1, True, False)
_edge_gather_16 = _make_edge_gather(K, False, True)


def _sc_emb_gather(tab, idx_flat):
    @functools.partial(
        pl.kernel,
        out_type=jax.ShapeDtypeStruct((B * K, D), jnp.float32),
        mesh=_mesh(),
        compiler_params=pltpu.CompilerParams(needs_layout_passes=False),
        scratch_types=[
            pltpu.VMEM((MCH, MCI), jnp.int32),
            pltpu.VMEM((MI, D), jnp.float32),
            pltpu.SemaphoreType.DMA,
        ],
    )
    def k(tab_hbm, idx_hbm, out_hbm, idx_v, rows_v, sem):
        wid = _wid()
        for ch in range(MCH):
            pltpu.sync_copy(idx_hbm.at[pl.ds(wid * MI + ch * MCI, MCI)],
                            idx_v.at[ch])
        copies = [
            pltpu.async_copy(tab_hbm.at[idx_v.at[ch]],
                             rows_v.at[pl.ds(ch * MCI, MCI)], sem)
            for ch in range(MCH)
        ]
        for c in copies:
            c.wait()
        pltpu.sync_copy(rows_v, out_hbm.at[pl.ds(wid * MI, MI)])

    return k(tab, idx_flat)


def _make_topk(N, with_prev):
    Bb = 128

    def body(sc_ref, pk_ref, *rest):
        if with_prev:
            ppk_ref, aent_ref, arel_ref, apk_ref, pf_ref, pn_ref, arp_ref = rest
        else:
            aent_ref, arel_ref, apk_ref = rest
        sc = sc_ref[...]
        pk = pk_ref[...]
        colid = lax.broadcasted_iota(jnp.int32, (Bb, N), 1)
        if with_prev:
            ppk = ppk_ref[...]
            jid = lax.broadcasted_iota(jnp.int32, (Bb, K), 1)
        apk_c, pf_c, ppk_c = [], [], []
        for _ in range(K):
            m = jnp.max(sc, axis=1, keepdims=True)
            eq = sc == m
            idx = jnp.min(jnp.where(eq, colid, N), axis=1, keepdims=True)
            hit = colid == idx
            apk_c.append(jnp.sum(jnp.where(hit, pk, 0), axis=1, keepdims=True))
            sc = jnp.where(hit, -1.0, sc)
            if with_prev:
                p = idx // NEI
                pf_c.append(p.astype(jnp.float32))
                ppk_c.append(jnp.sum(jnp.where(jid == p, ppk, 0),
                                     axis=1, keepdims=True))
        apk = jnp.concatenate(apk_c, axis=1)
        aent_ref[...] = apk >> 11
        arel_ref[...] = apk & 2047
        apk_ref[...] = apk
        if with_prev:
            pf_ref[...] = jnp.concatenate(pf_c, axis=1)
            psel = jnp.concatenate(ppk_c, axis=1)
            pn_ref[...] = psel >> 11
            arp_ref[...] = psel & 2047

    grid = (B // Bb,)
    bigspec = pl.BlockSpec((Bb, N), lambda i: (i, 0))
    kspec = pl.BlockSpec((Bb, K), lambda i: (i, 0))
    in_specs = [bigspec, bigspec] + ([kspec] if with_prev else [])
    n_out = 6 if with_prev else 3
    f32_outs = {3} if with_prev else set()
    out_shape = tuple(
        jax.ShapeDtypeStruct((B, K),
                             jnp.float32 if j in f32_outs else jnp.int32)
        for j in range(n_out)
    )
    return pl.pallas_call(
        body,
        grid=grid,
        in_specs=in_specs,
        out_specs=tuple([kspec] * n_out),
        out_shape=out_shape,
    )


_topk_1 = _make_topk(NEI, False)
_topk_16 = _make_topk(K * NEI, True)


def _gru_fused():
    Mb = 2048
    GB = Mb // K

    def body(x1_ref, x2_ref, x3_ref, p2_ref, p3_ref,
             wih_ref, whh_ref, bih_ref, bhh_ref,
             o1_ref, o2_ref, o3_ref):
        wih = wih_ref[...].astype(jnp.bfloat16)
        whh = whh_ref[...].astype(jnp.bfloat16)
        bih = bih_ref[...]
        bhh = bhh_ref[...]

        def sel(pe, p1):
            pe3 = pe.reshape(GB, K, D)
            h = jnp.zeros((Mb, D), jnp.float32)
            for j in range(K):
                src = lax.broadcast_in_dim(
                    pe3[:, j, :], (GB, K, D), (0, 2)).reshape(Mb, D)
                h = jnp.where(p1 == j, src, h)
            return h

        def gru_step(x, h):
            gi = lax.dot_general(x.astype(jnp.bfloat16), wih,
                                 (((1,), (1,)), ((), ())),
                                 preferred_element_type=jnp.float32) + bih
            if h is None:
                gh = bhh
            else:
                gh = lax.dot_general(h.astype(jnp.bfloat16), whh,
                                     (((1,), (1,)), ((), ())),
                                     preferred_element_type=jnp.float32) + bhh
            r = 1.0 / (1.0 + jnp.exp(-(gi[:, :D] + gh[:, :D])))
            z = 1.0 / (1.0 + jnp.exp(-(gi[:, D:2 * D] + gh[:, D:2 * D])))
            n = jnp.tanh(gi[:, 2 * D:] + r * gh[:, 2 * D:])
            if h is None:
                return (1.0 - z) * n
            return (1.0 - z) * n + z * h

        e1 = gru_step(x1_ref[...], None)
        o1_ref[...] = e1
        e2 = gru_step(x2_ref[...], sel(e1, p2_ref[...]))
        o2_ref[...] = e2
        e3 = gru_step(x3_ref[...], sel(e2, p3_ref[...]))
        o3_ref[...] = e3

    grid = ((B * K) // Mb,)
    xspec = pl.BlockSpec((Mb, D), lambda i: (i, 0))
    pspec = pl.BlockSpec((Mb, 1), lambda i: (i, 0))
    wspec = pl.BlockSpec((3 * D, D), lambda i: (0, 0))
    bspec = pl.BlockSpec((1, 3 * D), lambda i: (0, 0))
    eshape = jax.ShapeDtypeStruct((B * K, D), jnp.float32)
    return pl.pallas_call(
        body,
        grid=grid,
        in_specs=[xspec, xspec, xspec, pspec, pspec, wspec, wspec, bspec,
                  bspec],
        out_specs=(xspec, xspec, xspec),
        out_shape=(eshape, eshape, eshape),
    )


_gru_all = _gru_fused()


def kernel(support_tree_emb, support_rel, query_head, cos_rel_all, t_h, Train,
           rel_emb_table, edge_matrix, w_ih, w_hh, b_ih, b_hh):
    srel = support_rel.reshape(-1).astype(jnp.int32)
    qh = query_head.astype(jnp.int32)
    edge2d = edge_matrix.reshape(NE // 2, 2 * EW // 2)
    bih2 = b_ih.reshape(1, 3 * D)
    bhh2 = b_hh.reshape(1, 3 * D)

    sc1, pk1, rel_score = _edge_gather_1(edge2d, qh, cos_rel_all, srel)
    aim_ent1, aim_rel1, apk1 = _topk_1(sc1, pk1)

    def hop(aim_ent_p, aim_rel_p, apk_p):
        sc, pk, rel_e_p = _edge_gather_16(
            edge2d, aim_ent_p.reshape(-1), rel_score,
            rel_emb_table, aim_rel_p.reshape(-1))
        aent, arel, apk, pf, pn, arp = _topk_16(sc, pk, apk_p)
        return aent, arel, apk, rel_e_p, pf, pn, arp

    aim_ent2, aim_rel2, apk2, rel_e1, pf2, pn2, arp2 = hop(
        aim_ent1, aim_rel1, apk1)
    aim_ent3, aim_rel3, apk3, rel_e2, pf3, pn3, arp3 = hop(
        aim_ent2, aim_rel2, apk2)
    rel_e3 = _sc_emb_gather(rel_emb_table, aim_rel3.reshape(-1))
    emb1, emb2, emb3 = _gru_all(
        rel_e1, rel_e2, rel_e3,
        pf2.astype(jnp.int32).reshape(B * K, 1),
        pf3.astype(jnp.int32).reshape(B * K, 1),
        w_ih, w_hh, bih2, bhh2)

    tree_node = jnp.stack([aim_ent1, aim_ent2, aim_ent3], 1)
    tree_emb_all = jnp.stack(
        [emb1.reshape(B, K, D), emb2.reshape(B, K, D), emb3.reshape(B, K, D)], 1)
    parent_index = jnp.stack(
        [pf2, pf3, jnp.tile(jnp.arange(K, dtype=jnp.float32)[None, :], (B, 1))], 1)
    parent_node = jnp.stack([jnp.tile(qh[:, None], (1, K)), pn2, pn3], 1)
    aim_rel_all = jnp.stack([arp2, arp3, aim_rel3], 1)
    return tree_node, tree_emb_all, parent_index, parent_node, aim_rel_all


# speedup vs baseline: 6.0991x; 6.0991x over previous
import functools

import jax
import jax.numpy as jnp
from jax import lax
from jax.experimental import pallas as pl
from jax.experimental.pallas import tpu as pltpu
from jax.experimental.pallas import tpu_sc as plsc

D = 128
NEI = 32
K = 16
B = 1024
R = 2000
NE = 50000
S = 16
RP = 2048
EW = 128
NC = 2
NS = 16
NW = NC * NS
LANES = 16

MI = (B * K) // NW
MCI = 128
MCH = MI // MCI


def _mesh():
    return plsc.VectorSubcoreMesh(core_axis_name="c", subcore_axis_name="s")


def _wid():
    return lax.axis_index("s") * NC + lax.axis_index("c")


def _dg16(vec, idx):
    return lax.gather(
        vec, idx[:, None],
        lax.GatherDimensionNumbers(
            offset_dims=(), collapsed_slice_dims=(0,), start_index_map=(0,)),
        (1,), mode=lax.GatherScatterMode.PROMISE_IN_BOUNDS)


def _make_edge_gather(C, first, with_emb):
    WB = B // NW
    NI = WB * C
    CI = min(128, NI)
    NCH = NI // CI
    BC = CI // C
    N = C * NEI

    out_type = [
        jax.ShapeDtypeStruct((B, N), jnp.float32),
        jax.ShapeDtypeStruct((B, N), jnp.int32),
    ]
    scratch = [
        pltpu.VMEM((R,), jnp.float32),
        pltpu.VMEM((NCH, CI), jnp.int32),
        pltpu.VMEM((NI, EW), jnp.int32),
        pltpu.VMEM((BC, N), jnp.float32),
        pltpu.VMEM((BC, N), jnp.int32),
        pltpu.SemaphoreType.DMA,
    ]
    if first:
        out_type.append(jax.ShapeDtypeStruct((R,), jnp.float32))
        scratch = [pltpu.VMEM((S,), jnp.int32),
                   pltpu.VMEM((S, R), jnp.float32),
                   pltpu.SemaphoreType.DMA] + scratch
    if with_emb:
        out_type.append(jax.ShapeDtypeStruct((B * K, D), jnp.float32))
        scratch += [pltpu.VMEM((MCH, MCI), jnp.int32),
                    pltpu.VMEM((MCI, D), jnp.float32),
                    pltpu.VMEM((MCI, D), jnp.float32),
                    pltpu.SemaphoreType.DMA,
                    pltpu.SemaphoreType.DMA]

    @functools.partial(
        pl.kernel,
        out_type=tuple(out_type),
        mesh=_mesh(),
        compiler_params=pltpu.CompilerParams(needs_layout_passes=False),
        scratch_types=scratch,
    )
    def k(*refs):
        it = iter(refs)
        edge_hbm, cur_hbm = next(it), next(it)
        if first:
            cos_hbm, srel_hbm = next(it), next(it)
        else:
            rs_hbm = next(it)
        if with_emb:
            emtab_hbm, erel_hbm = next(it), next(it)
        osc_hbm, opk_hbm = next(it), next(it)
        if first:
            rs_hbm = next(it)
        if with_emb:
            emb_hbm = next(it)
        if first:
            srel_v, cos_v, csem = next(it), next(it), next(it)
        tab_v, idx_v, rows_v, osc_v, opk_v, sem = (
            next(it), next(it), next(it), next(it), next(it), next(it))
        if with_emb:
            midx_v, mrows0_v, mrows1_v, msem0, msem1 = (
                next(it), next(it), next(it), next(it), next(it))
        wid = _wid()

        for ch in range(NCH):
            pltpu.sync_copy(cur_hbm.at[pl.ds(wid * NI + ch * CI, CI)],
                            idx_v.at[ch])
        ecopies = [
            pltpu.async_copy(edge_hbm.at[idx_v.at[ch]],
                             rows_v.at[pl.ds(ch * CI, CI)], sem)
            for ch in range(NCH)
        ]
        if with_emb:
            for ch in range(MCH):
                pltpu.sync_copy(erel_hbm.at[pl.ds(wid * MI + ch * MCI, MCI)],
                                midx_v.at[ch])
            mbufs = [mrows0_v, mrows1_v]
            msems = [msem0, msem1]
            mcopies = [
                pltpu.async_copy(emtab_hbm.at[midx_v.at[ch]],
                                 mbufs[ch % 2], msems[ch % 2])
                for ch in range(2)
            ]
        if first:
            pltpu.sync_copy(srel_hbm, srel_v)
            srel_vec = srel_v[...]
            slane = lax.iota(jnp.int32, LANES)
            scopies = []
            for s in range(S):
                row = jnp.max(jnp.where(slane == s, srel_vec, 0))
                scopies.append(
                    pltpu.async_copy(cos_hbm.at[row], cos_v.at[s], csem))
            for c in scopies:
                c.wait()

            def tbody(j, carry):
                sl = pl.ds(j * LANES, LANES)
                m = cos_v[0, sl]
                for s in range(1, S):
                    m = jnp.maximum(m, cos_v[s, sl])
                tab_v[sl] = m
                return carry

            lax.fori_loop(0, R // LANES, tbody, 0)

            @pl.when(wid == 0)
            def _():
                pltpu.sync_copy(tab_v, rs_hbm)
        else:
            pltpu.sync_copy(rs_hbm, tab_v)

        lane = lax.iota(jnp.int32, LANES)
        pat_e = (lane & 7) * 2
        pat_o = pat_e + 1
        lo = lane < 8
        for ch in range(NCH):
            ecopies[ch].wait()

            def body(brow, carry):
                for c in range(C):
                    r = ch * CI + brow * C + c
                    for v2 in range(2):
                        a = rows_v[r, pl.ds(v2 * 2 * LANES, LANES)]
                        b = rows_v[r, pl.ds(v2 * 2 * LANES + LANES, LANES)]
                        entv = jnp.where(lo, _dg16(a, pat_e), _dg16(b, pat_e))
                        relv = jnp.where(lo, _dg16(a, pat_o), _dg16(b, pat_o))
                        scv = plsc.load_gather(tab_v, [relv])
                        o = pl.ds(c * NEI + v2 * LANES, LANES)
                        osc_v[brow, o] = scv
                        opk_v[brow, o] = entv * 2048 + relv
                return carry

            lax.fori_loop(0, BC, body, 0)
            ob = wid * WB + ch * BC
            pltpu.sync_copy(osc_v, osc_hbm.at[pl.ds(ob, BC)])
            pltpu.sync_copy(opk_v, opk_hbm.at[pl.ds(ob, BC)])

        if with_emb:
            for ch in range(MCH):
                mcopies[ch].wait()
                pltpu.sync_copy(
                    mbufs[ch % 2],
                    emb_hbm.at[pl.ds(wid * MI + ch * MCI, MCI)])
                if ch + 2 < MCH:
                    mcopies.append(pltpu.async_copy(
                        emtab_hbm.at[midx_v.at[ch + 2]],
                        mbufs[ch % 2], msems[ch % 2]))

    return k


_edge_gather_1 = _make_edge_gather(1, True, False)
_edge_gather_16 = _make_edge_gather(K, False, True)


def _sc_emb_gather(tab, idx_flat):
    @functools.partial(
        pl.kernel,
        out_type=jax.ShapeDtypeStruct((B * K, D), jnp.float32),
        mesh=_mesh(),
        compiler_params=pltpu.CompilerParams(needs_layout_passes=False),
        scratch_types=[
            pltpu.VMEM((MCH, MCI), jnp.int32),
            pltpu.VMEM((MI, D), jnp.float32),
            pltpu.SemaphoreType.DMA,
        ],
    )
    def k(tab_hbm, idx_hbm, out_hbm, idx_v, rows_v, sem):
        wid = _wid()
        for ch in range(MCH):
            pltpu.sync_copy(idx_hbm.at[pl.ds(wid * MI + ch * MCI, MCI)],
                            idx_v.at[ch])
        copies = [
            pltpu.async_copy(tab_hbm.at[idx_v.at[ch]],
                             rows_v.at[pl.ds(ch * MCI, MCI)], sem)
            for ch in range(MCH)
        ]
        for c in copies:
            c.wait()
        pltpu.sync_copy(rows_v, out_hbm.at[pl.ds(wid * MI, MI)])

    return k(tab, idx_flat)


def _make_topk(N, with_prev):
    Bb = 1024 if N <= NEI else 256

    def body(sc_ref, pk_ref, *rest):
        if with_prev:
            ppk_ref, aent_ref, arel_ref, apk_ref, pf_ref, pn_ref, arp_ref = rest
        else:
            aent_ref, arel_ref, apk_ref = rest
        sc = sc_ref[...]
        pk = pk_ref[...]
        colid = lax.broadcasted_iota(jnp.int32, (Bb, N), 1)
        if with_prev:
            ppk = ppk_ref[...]
            jid = lax.broadcasted_iota(jnp.int32, (Bb, K), 1)
        apk_c, pf_c, ppk_c = [], [], []
        for _ in range(K):
            m = jnp.max(sc, axis=1, keepdims=True)
            eq = sc == m
            idx = jnp.min(jnp.where(eq, colid, N), axis=1, keepdims=True)
            hit = colid == idx
            apk_c.append(jnp.sum(jnp.where(hit, pk, 0), axis=1, keepdims=True))
            sc = jnp.where(hit, -1.0, sc)
            if with_prev:
                p = idx // NEI
                pf_c.append(p.astype(jnp.float32))
                ppk_c.append(jnp.sum(jnp.where(jid == p, ppk, 0),
                                     axis=1, keepdims=True))
        apk = jnp.concatenate(apk_c, axis=1)
        aent_ref[...] = apk >> 11
        arel_ref[...] = apk & 2047
        apk_ref[...] = apk
        if with_prev:
            pf_ref[...] = jnp.concatenate(pf_c, axis=1)
            psel = jnp.concatenate(ppk_c, axis=1)
            pn_ref[...] = psel >> 11
            arp_ref[...] = psel & 2047

    grid = (B // Bb,)
    bigspec = pl.BlockSpec((Bb, N), lambda i: (i, 0))
    kspec = pl.BlockSpec((Bb, K), lambda i: (i, 0))
    in_specs = [bigspec, bigspec] + ([kspec] if with_prev else [])
    n_out = 6 if with_prev else 3
    f32_outs = {3} if with_prev else set()
    out_shape = tuple(
        jax.ShapeDtypeStruct((B, K),
                             jnp.float32 if j in f32_outs else jnp.int32)
        for j in range(n_out)
    )
    return pl.pallas_call(
        body,
        grid=grid,
        in_specs=in_specs,
        out_specs=tuple([kspec] * n_out),
        out_shape=out_shape,
    )


_topk_1 = _make_topk(NEI, False)
_topk_16 = _make_topk(K * NEI, True)


def _gru_fused():
    Mb = 2048
    GB = Mb // K

    def body(x1_ref, x2_ref, x3_ref, p2_ref, p3_ref,
             wih_ref, whh_ref, bih_ref, bhh_ref,
             o1_ref, o2_ref, o3_ref):
        wih = wih_ref[...].astype(jnp.bfloat16)
        whh = whh_ref[...].astype(jnp.bfloat16)
        bih = bih_ref[...]
        bhh = bhh_ref[...]

        def sel(pe, p1):
            pe3 = pe.reshape(GB, K, D)
            h = jnp.zeros((Mb, D), jnp.float32)
            for j in range(K):
                src = lax.broadcast_in_dim(
                    pe3[:, j, :], (GB, K, D), (0, 2)).reshape(Mb, D)
                h = jnp.where(p1 == j, src, h)
            return h

        def gru_step(x, h):
            gi = lax.dot_general(x.astype(jnp.bfloat16), wih,
                                 (((1,), (1,)), ((), ())),
                                 preferred_element_type=jnp.float32) + bih
            if h is None:
                gh = bhh
            else:
                gh = lax.dot_general(h.astype(jnp.bfloat16), whh,
                                     (((1,), (1,)), ((), ())),
                                     preferred_element_type=jnp.float32) + bhh
            r = 1.0 / (1.0 + jnp.exp(-(gi[:, :D] + gh[:, :D])))
            z = 1.0 / (1.0 + jnp.exp(-(gi[:, D:2 * D] + gh[:, D:2 * D])))
            n = jnp.tanh(gi[:, 2 * D:] + r * gh[:, 2 * D:])
            if h is None:
                return (1.0 - z) * n
            return (1.0 - z) * n + z * h

        e1 = gru_step(x1_ref[...], None)
        o1_ref[...] = e1
        e2 = gru_step(x2_ref[...], sel(e1, p2_ref[...]))
        o2_ref[...] = e2
        e3 = gru_step(x3_ref[...], sel(e2, p3_ref[...]))
        o3_ref[...] = e3

    grid = ((B * K) // Mb,)
    xspec = pl.BlockSpec((Mb, D), lambda i: (i, 0))
    pspec = pl.BlockSpec((Mb, 1), lambda i: (i, 0))
    wspec = pl.BlockSpec((3 * D, D), lambda i: (0, 0))
    bspec = pl.BlockSpec((1, 3 * D), lambda i: (0, 0))
    eshape = jax.ShapeDtypeStruct((B * K, D), jnp.float32)
    return pl.pallas_call(
        body,
        grid=grid,
        in_specs=[xspec, xspec, xspec, pspec, pspec, wspec, wspec, bspec,
                  bspec],
        out_specs=(xspec, xspec, xspec),
        out_shape=(eshape, eshape, eshape),
    )


_gru_all = _gru_fused()


def kernel(support_tree_emb, support_rel, query_head, cos_rel_all, t_h, Train,
           rel_emb_table, edge_matrix, w_ih, w_hh, b_ih, b_hh):
    srel = support_rel.reshape(-1).astype(jnp.int32)
    qh = query_head.astype(jnp.int32)
    edge2d = jnp.pad(edge_matrix.reshape(NE, 2 * NEI),
                     ((0, 0), (0, EW - 2 * NEI)))
    bih2 = b_ih.reshape(1, 3 * D)
    bhh2 = b_hh.reshape(1, 3 * D)

    sc1, pk1, rel_score = _edge_gather_1(edge2d, qh, cos_rel_all, srel)
    aim_ent1, aim_rel1, apk1 = _topk_1(sc1, pk1)

    def hop(aim_ent_p, aim_rel_p, apk_p):
        sc, pk, rel_e_p = _edge_gather_16(
            edge2d, aim_ent_p.reshape(-1), rel_score,
            rel_emb_table, aim_rel_p.reshape(-1))
        aent, arel, apk, pf, pn, arp = _topk_16(sc, pk, apk_p)
        return aent, arel, apk, rel_e_p, pf, pn, arp

    aim_ent2, aim_rel2, apk2, rel_e1, pf2, pn2, arp2 = hop(
        aim_ent1, aim_rel1, apk1)
    aim_ent3, aim_rel3, apk3, rel_e2, pf3, pn3, arp3 = hop(
        aim_ent2, aim_rel2, apk2)
    rel_e3 = _sc_emb_gather(rel_emb_table, aim_rel3.reshape(-1))
    emb1, emb2, emb3 = _gru_all(
        rel_e1, rel_e2, rel_e3,
        pf2.astype(jnp.int32).reshape(B * K, 1),
        pf3.astype(jnp.int32).reshape(B * K, 1),
        w_ih, w_hh, bih2, bhh2)

    tree_node = jnp.stack([aim_ent1, aim_ent2, aim_ent3], 1)
    tree_emb_all = jnp.stack(
        [emb1.reshape(B, K, D), emb2.reshape(B, K, D), emb3.reshape(B, K, D)], 1)
    parent_index = jnp.stack(
        [pf2, pf3, jnp.tile(jnp.arange(K, dtype=jnp.float32)[None, :], (B, 1))], 1)
    parent_node = jnp.stack([jnp.tile(qh[:, None], (1, K)), pn2, pn3], 1)
    aim_rel_all = jnp.stack([arp2, arp3, aim_rel3], 1)
    return tree_node, tree_emb_all, parent_index, parent_node, aim_rel_all


# speedup vs baseline: 6.1048x; 1.0009x over previous
import functools

import jax
import jax.numpy as jnp
from jax import lax
from jax.experimental import pallas as pl
from jax.experimental.pallas import tpu as pltpu
from jax.experimental.pallas import tpu_sc as plsc

D = 128
NEI = 32
K = 16
B = 1024
R = 2000
NE = 50000
S = 16
RP = 2048
EW = 128
NC = 2
NS = 16
NW = NC * NS
LANES = 16

MI = (B * K) // NW
MCI = 128
MCH = MI // MCI


def _mesh():
    return plsc.VectorSubcoreMesh(core_axis_name="c", subcore_axis_name="s")


def _wid():
    return lax.axis_index("s") * NC + lax.axis_index("c")


def _dg16(vec, idx):
    return lax.gather(
        vec, idx[:, None],
        lax.GatherDimensionNumbers(
            offset_dims=(), collapsed_slice_dims=(0,), start_index_map=(0,)),
        (1,), mode=lax.GatherScatterMode.PROMISE_IN_BOUNDS)


def _make_edge_gather(C, first, with_emb):
    WB = B // NW
    NI = WB * C
    CI = min(128, NI)
    NCH = NI // CI
    BC = CI // C
    N = C * NEI

    out_type = [
        jax.ShapeDtypeStruct((B, N), jnp.float32),
        jax.ShapeDtypeStruct((B, N), jnp.int32),
    ]
    scratch = [
        pltpu.VMEM((R,), jnp.float32),
        pltpu.VMEM((NCH, CI), jnp.int32),
        pltpu.VMEM((NI, EW), jnp.int32),
        pltpu.VMEM((BC, N), jnp.float32),
        pltpu.VMEM((BC, N), jnp.int32),
        pltpu.SemaphoreType.DMA,
    ]
    if first:
        out_type.append(jax.ShapeDtypeStruct((R,), jnp.float32))
        scratch = [pltpu.VMEM((S,), jnp.int32),
                   pltpu.VMEM((S, R), jnp.float32),
                   pltpu.SemaphoreType.DMA] + scratch
    if with_emb:
        out_type.append(jax.ShapeDtypeStruct((B * K, D), jnp.float32))
        scratch += [pltpu.VMEM((MCH, MCI), jnp.int32),
                    pltpu.VMEM((MCI, D), jnp.float32),
                    pltpu.VMEM((MCI, D), jnp.float32),
                    pltpu.SemaphoreType.DMA,
                    pltpu.SemaphoreType.DMA]

    @functools.partial(
        pl.kernel,
        out_type=tuple(out_type),
        mesh=_mesh(),
        compiler_params=pltpu.CompilerParams(needs_layout_passes=False),
        scratch_types=scratch,
    )
    def k(*refs):
        it = iter(refs)
        edge_hbm, cur_hbm = next(it), next(it)
        if first:
            cos_hbm, srel_hbm = next(it), next(it)
        else:
            rs_hbm = next(it)
        if with_emb:
            emtab_hbm, erel_hbm = next(it), next(it)
        osc_hbm, opk_hbm = next(it), next(it)
        if first:
            rs_hbm = next(it)
        if with_emb:
            emb_hbm = next(it)
        if first:
            srel_v, cos_v, csem = next(it), next(it), next(it)
        tab_v, idx_v, rows_v, osc_v, opk_v, sem = (
            next(it), next(it), next(it), next(it), next(it), next(it))
        if with_emb:
            midx_v, mrows0_v, mrows1_v, msem0, msem1 = (
                next(it), next(it), next(it), next(it), next(it))
        wid = _wid()

        for ch in range(NCH):
            pltpu.sync_copy(cur_hbm.at[pl.ds(wid * NI + ch * CI, CI)],
                            idx_v.at[ch])
        ecopies = [
            pltpu.async_copy(edge_hbm.at[idx_v.at[ch]],
                             rows_v.at[pl.ds(ch * CI, CI)], sem)
            for ch in range(NCH)
        ]
        if with_emb:
            for ch in range(MCH):
                pltpu.sync_copy(erel_hbm.at[pl.ds(wid * MI + ch * MCI, MCI)],
                                midx_v.at[ch])
            mbufs = [mrows0_v, mrows1_v]
            msems = [msem0, msem1]
            mcopies = [
                pltpu.async_copy(emtab_hbm.at[midx_v.at[ch]],
                                 mbufs[ch % 2], msems[ch % 2])
                for ch in range(2)
            ]
        if first:
            pltpu.sync_copy(srel_hbm, srel_v)
            srel_vec = srel_v[...]
            slane = lax.iota(jnp.int32, LANES)
            scopies = []
            for s in range(S):
                row = jnp.max(jnp.where(slane == s, srel_vec, 0))
                scopies.append(
                    pltpu.async_copy(cos_hbm.at[row], cos_v.at[s], csem))
            for c in scopies:
                c.wait()

            def tbody(j, carry):
                sl = pl.ds(j * LANES, LANES)
                m = cos_v[0, sl]
                for s in range(1, S):
                    m = jnp.maximum(m, cos_v[s, sl])
                tab_v[sl] = m
                return carry

            lax.fori_loop(0, R // LANES, tbody, 0)

            @pl.when(wid == 0)
            def _():
                pltpu.sync_copy(tab_v, rs_hbm)
        else:
            pltpu.sync_copy(rs_hbm, tab_v)

        lane = lax.iota(jnp.int32, LANES)
        pat_e = (lane & 7) * 2
        pat_o = pat_e + 1
        lo = lane < 8
        for ch in range(NCH):
            ecopies[ch].wait()

            def body(brow, carry):
                for c in range(C):
                    r = ch * CI + brow * C + c
                    for v2 in range(2):
                        a = rows_v[r, pl.ds(v2 * 2 * LANES, LANES)]
                        b = rows_v[r, pl.ds(v2 * 2 * LANES + LANES, LANES)]
                        entv = jnp.where(lo, _dg16(a, pat_e), _dg16(b, pat_e))
                        relv = jnp.where(lo, _dg16(a, pat_o), _dg16(b, pat_o))
                        scv = plsc.load_gather(tab_v, [relv])
                        o = pl.ds(c * NEI + v2 * LANES, LANES)
                        osc_v[brow, o] = scv
                        opk_v[brow, o] = entv * 2048 + relv
                return carry

            lax.fori_loop(0, BC, body, 0)
            ob = wid * WB + ch * BC
            pltpu.sync_copy(osc_v, osc_hbm.at[pl.ds(ob, BC)])
            pltpu.sync_copy(opk_v, opk_hbm.at[pl.ds(ob, BC)])

        if with_emb:
            for ch in range(MCH):
                mcopies[ch].wait()
                pltpu.sync_copy(
                    mbufs[ch % 2],
                    emb_hbm.at[pl.ds(wid * MI + ch * MCI, MCI)])
                if ch + 2 < MCH:
                    mcopies.append(pltpu.async_copy(
                        emtab_hbm.at[midx_v.at[ch + 2]],
                        mbufs[ch % 2], msems[ch % 2]))

    return k


_edge_gather_1 = _make_edge_gather(1, True, False)
_edge_gather_16 = _make_edge_gather(K, False, True)


def _sc_emb_gather(tab, idx_flat):
    @functools.partial(
        pl.kernel,
        out_type=jax.ShapeDtypeStruct((B * K, D), jnp.float32),
        mesh=_mesh(),
        compiler_params=pltpu.CompilerParams(needs_layout_passes=False),
        scratch_types=[
            pltpu.VMEM((MCH, MCI), jnp.int32),
            pltpu.VMEM((MI, D), jnp.float32),
            pltpu.SemaphoreType.DMA,
        ],
    )
    def k(tab_hbm, idx_hbm, out_hbm, idx_v, rows_v, sem):
        wid = _wid()
        for ch in range(MCH):
            pltpu.sync_copy(idx_hbm.at[pl.ds(wid * MI + ch * MCI, MCI)],
                            idx_v.at[ch])
        copies = [
            pltpu.async_copy(tab_hbm.at[idx_v.at[ch]],
                             rows_v.at[pl.ds(ch * MCI, MCI)], sem)
            for ch in range(MCH)
        ]
        for c in copies:
            c.wait()
        pltpu.sync_copy(rows_v, out_hbm.at[pl.ds(wid * MI, MI)])

    return k(tab, idx_flat)


def _make_topk(N, with_prev):
    Bb = 1024 if N <= NEI else 512

    def body(sc_ref, pk_ref, *rest):
        if with_prev:
            ppk_ref, aent_ref, arel_ref, apk_ref, pf_ref, pn_ref, arp_ref = rest
        else:
            aent_ref, arel_ref, apk_ref = rest
        sc = sc_ref[...]
        pk = pk_ref[...]
        colid = lax.broadcasted_iota(jnp.int32, (Bb, N), 1)
        if with_prev:
            ppk = ppk_ref[...]
            jid = lax.broadcasted_iota(jnp.int32, (Bb, K), 1)
        apk_c, pf_c, ppk_c = [], [], []
        for _ in range(K):
            m = jnp.max(sc, axis=1, keepdims=True)
            eq = sc == m
            idx = jnp.min(jnp.where(eq, colid, N), axis=1, keepdims=True)
            hit = colid == idx
            apk_c.append(jnp.sum(jnp.where(hit, pk, 0), axis=1, keepdims=True))
            sc = jnp.where(hit, -1.0, sc)
            if with_prev:
                p = idx // NEI
                pf_c.append(p.astype(jnp.float32))
                ppk_c.append(jnp.sum(jnp.where(jid == p, ppk, 0),
                                     axis=1, keepdims=True))
        apk = jnp.concatenate(apk_c, axis=1)
        aent_ref[...] = apk >> 11
        arel_ref[...] = apk & 2047
        apk_ref[...] = apk
        if with_prev:
            pf_ref[...] = jnp.concatenate(pf_c, axis=1)
            psel = jnp.concatenate(ppk_c, axis=1)
            pn_ref[...] = psel >> 11
            arp_ref[...] = psel & 2047

    grid = (B // Bb,)
    bigspec = pl.BlockSpec((Bb, N), lambda i: (i, 0))
    kspec = pl.BlockSpec((Bb, K), lambda i: (i, 0))
    in_specs = [bigspec, bigspec] + ([kspec] if with_prev else [])
    n_out = 6 if with_prev else 3
    f32_outs = {3} if with_prev else set()
    out_shape = tuple(
        jax.ShapeDtypeStruct((B, K),
                             jnp.float32 if j in f32_outs else jnp.int32)
        for j in range(n_out)
    )
    return pl.pallas_call(
        body,
        grid=grid,
        in_specs=in_specs,
        out_specs=tuple([kspec] * n_out),
        out_shape=out_shape,
    )


_topk_1 = _make_topk(NEI, False)
_topk_16 = _make_topk(K * NEI, True)


def _gru_fused():
    Mb = 4096
    GB = Mb // K

    def body(x1_ref, x2_ref, x3_ref, p2_ref, p3_ref,
             wih_ref, whh_ref, bih_ref, bhh_ref,
             o1_ref, o2_ref, o3_ref):
        wih = wih_ref[...].astype(jnp.bfloat16)
        whh = whh_ref[...].astype(jnp.bfloat16)
        bih = bih_ref[...]
        bhh = bhh_ref[...]

        def sel(pe, p1):
            pe3 = pe.reshape(GB, K, D)
            h = jnp.zeros((Mb, D), jnp.float32)
            for j in range(K):
                src = lax.broadcast_in_dim(
                    pe3[:, j, :], (GB, K, D), (0, 2)).reshape(Mb, D)
                h = jnp.where(p1 == j, src, h)
            return h

        def gru_step(x, h):
            gi = lax.dot_general(x.astype(jnp.bfloat16), wih,
                                 (((1,), (1,)), ((), ())),
                                 preferred_element_type=jnp.float32) + bih
            if h is None:
                gh = bhh
            else:
                gh = lax.dot_general(h.astype(jnp.bfloat16), whh,
                                     (((1,), (1,)), ((), ())),
                                     preferred_element_type=jnp.float32) + bhh
            r = 1.0 / (1.0 + jnp.exp(-(gi[:, :D] + gh[:, :D])))
            z = 1.0 / (1.0 + jnp.exp(-(gi[:, D:2 * D] + gh[:, D:2 * D])))
            n = jnp.tanh(gi[:, 2 * D:] + r * gh[:, 2 * D:])
            if h is None:
                return (1.0 - z) * n
            return (1.0 - z) * n + z * h

        e1 = gru_step(x1_ref[...], None)
        o1_ref[...] = e1
        e2 = gru_step(x2_ref[...], sel(e1, p2_ref[...]))
        o2_ref[...] = e2
        e3 = gru_step(x3_ref[...], sel(e2, p3_ref[...]))
        o3_ref[...] = e3

    grid = ((B * K) // Mb,)
    xspec = pl.BlockSpec((Mb, D), lambda i: (i, 0))
    pspec = pl.BlockSpec((Mb, 1), lambda i: (i, 0))
    wspec = pl.BlockSpec((3 * D, D), lambda i: (0, 0))
    bspec = pl.BlockSpec((1, 3 * D), lambda i: (0, 0))
    eshape = jax.ShapeDtypeStruct((B * K, D), jnp.float32)
    return pl.pallas_call(
        body,
        grid=grid,
        in_specs=[xspec, xspec, xspec, pspec, pspec, wspec, wspec, bspec,
                  bspec],
        out_specs=(xspec, xspec, xspec),
        out_shape=(eshape, eshape, eshape),
    )


_gru_all = _gru_fused()


def kernel(support_tree_emb, support_rel, query_head, cos_rel_all, t_h, Train,
           rel_emb_table, edge_matrix, w_ih, w_hh, b_ih, b_hh):
    srel = support_rel.reshape(-1).astype(jnp.int32)
    qh = query_head.astype(jnp.int32)
    edge2d = jnp.pad(edge_matrix.reshape(NE, 2 * NEI),
                     ((0, 0), (0, EW - 2 * NEI)))
    bih2 = b_ih.reshape(1, 3 * D)
    bhh2 = b_hh.reshape(1, 3 * D)

    sc1, pk1, rel_score = _edge_gather_1(edge2d, qh, cos_rel_all, srel)
    aim_ent1, aim_rel1, apk1 = _topk_1(sc1, pk1)

    def hop(aim_ent_p, aim_rel_p, apk_p):
        sc, pk, rel_e_p = _edge_gather_16(
            edge2d, aim_ent_p.reshape(-1), rel_score,
            rel_emb_table, aim_rel_p.reshape(-1))
        aent, arel, apk, pf, pn, arp = _topk_16(sc, pk, apk_p)
        return aent, arel, apk, rel_e_p, pf, pn, arp

    aim_ent2, aim_rel2, apk2, rel_e1, pf2, pn2, arp2 = hop(
        aim_ent1, aim_rel1, apk1)
    aim_ent3, aim_rel3, apk3, rel_e2, pf3, pn3, arp3 = hop(
        aim_ent2, aim_rel2, apk2)
    rel_e3 = _sc_emb_gather(rel_emb_table, aim_rel3.reshape(-1))
    emb1, emb2, emb3 = _gru_all(
        rel_e1, rel_e2, rel_e3,
        pf2.astype(jnp.int32).reshape(B * K, 1),
        pf3.astype(jnp.int32).reshape(B * K, 1),
        w_ih, w_hh, bih2, bhh2)

    tree_node = jnp.stack([aim_ent1, aim_ent2, aim_ent3], 1)
    tree_emb_all = jnp.stack(
        [emb1.reshape(B, K, D), emb2.reshape(B, K, D), emb3.reshape(B, K, D)], 1)
    parent_index = jnp.stack(
        [pf2, pf3, jnp.tile(jnp.arange(K, dtype=jnp.float32)[None, :], (B, 1))], 1)
    parent_node = jnp.stack([jnp.tile(qh[:, None], (1, K)), pn2, pn3], 1)
    aim_rel_all = jnp.stack([arp2, arp3, aim_rel3], 1)
    return tree_node, tree_emb_all, parent_index, parent_node, aim_rel_all
